# Initial kernel scaffold; baseline (speedup 1.0000x reference)
import jax
import jax.numpy as jnp
from jax.experimental import pallas as pl


def kernel(node_id, edge_index, img_h, txt_h, emb_table, W_fc, a_attn):
    raise NotImplementedError("write your pallas kernel here")


# speedup vs baseline: 15.5378x; 15.5378x over previous
import functools

import jax
import jax.numpy as jnp
from jax import lax
from jax.experimental import pallas as pl
from jax.experimental.pallas import tpu as pltpu
from jax.experimental.pallas import tpu_sc as plsc

N = 10000
E = 320000
D = 128
NC = 2
NS = 16
NT = NC * NS
ET = E // NT
N_PAD = 10240
TN = N_PAD // NS
AR = 125
AW = 80
BR = ET // 16
BLK = 1000


def _prep_tc(h_ref, wt_ref, a_ref, z_ref, s_ref, m_ref):
    z = jnp.dot(h_ref[...], wt_ref[...], preferred_element_type=jnp.float32)
    z_ref[...] = z
    s = jnp.dot(z, a_ref[...], preferred_element_type=jnp.float32)
    s_ref[...] = s

    @pl.when(pl.program_id(0) == 0)
    def _():
        m_ref[...] = jnp.full((1, 2), -3.4e38, jnp.float32)

    m_ref[...] = jnp.maximum(m_ref[...], jnp.max(s, axis=0, keepdims=True))


def _finish_tc(n0_ref, n1_ref, d0_ref, d1_ref, o_ref):
    den = jnp.maximum(d0_ref[...] + d1_ref[...], 1e-30)
    o_ref[...] = (n0_ref[...] + n1_ref[...]) / den


_SC_MESH = plsc.VectorSubcoreMesh(core_axis_name="c", subcore_axis_name="s")
_SC_PARAMS = pltpu.CompilerParams(needs_layout_passes=False,
                                  use_tc_tiling_on_sc=False)


@functools.partial(
    pl.kernel,
    mesh=_SC_MESH,
    out_type=[
        jax.ShapeDtypeStruct((NT, AR, AW), jnp.float32),
        jax.ShapeDtypeStruct((NC, N_PAD), jnp.float32),
    ],
    scratch_types=[
        pltpu.VMEM((N,), jnp.float32),
        pltpu.VMEM((N,), jnp.float32),
        pltpu.VMEM((AR, AW), jnp.int32),
        pltpu.VMEM((AR, AW), jnp.int32),
        pltpu.VMEM((AR, AW), jnp.float32),
        pltpu.VMEM((16,), jnp.float32),
        pltpu.VMEM((TN,), jnp.float32),
        pltpu.VMEM_SHARED((N_PAD,), jnp.float32),
    ],
    compiler_params=_SC_PARAMS,
)
def _edge_sc(s1_hbm, s2_hbm, src_hbm, dst_hbm, shift_hbm,
             p_hbm, den_hbm,
             s1v, s2v, srcv, dstv, pv, shiftv, zbuf, den_sp):
    c = lax.axis_index("c")
    s = lax.axis_index("s")
    tile = c * NS + s
    nbase = s * TN

    pltpu.sync_copy(s1_hbm, s1v)
    pltpu.sync_copy(s2_hbm, s2v)
    pltpu.sync_copy(src_hbm.at[tile], srcv)
    pltpu.sync_copy(dst_hbm.at[tile], dstv)
    pltpu.sync_copy(shift_hbm, shiftv)
    shift = shiftv[...]

    zero = jnp.zeros((16,), jnp.float32)

    def zloop(i, _):
        zbuf[pl.ds(i * 16, 16)] = zero
        return ()

    lax.fori_loop(0, TN // 16, zloop, ())
    pltpu.sync_copy(zbuf, den_sp.at[pl.ds(nbase, TN)])
    plsc.subcore_barrier()

    def erow(j, _):
        for k in range(AW // 16):
            si = srcv[j, pl.ds(k * 16, 16)]
            di = dstv[j, pl.ds(k * 16, 16)]
            g1 = plsc.load_gather(s1v, [si])
            g2 = plsc.load_gather(s2v, [di])
            x = g1 + g2
            e = jnp.where(x >= 0.0, x, 0.01 * x)
            pv[j, pl.ds(k * 16, 16)] = jnp.exp(e - shift)
        pltpu.sync_copy(pv.at[j], den_sp.at[dstv.at[j]], add=True)
        return ()

    lax.fori_loop(0, AR, erow, ())
    pltpu.sync_copy(pv, p_hbm.at[tile])
    plsc.subcore_barrier()
    pltpu.sync_copy(den_sp.at[pl.ds(nbase, TN)], zbuf)
    pltpu.sync_copy(zbuf, den_hbm.at[c, pl.ds(nbase, TN)])


def _bcast_lane(v, r):
    dn = lax.GatherDimensionNumbers(offset_dims=(), collapsed_slice_dims=(0,),
                                    start_index_map=(0,))
    return lax.gather(v, jnp.full((16, 1), r, jnp.int32), dn, (1,),
                      mode=lax.GatherScatterMode.PROMISE_IN_BOUNDS)


D2 = D // 2


@functools.partial(
    pl.kernel,
    mesh=_SC_MESH,
    out_type=jax.ShapeDtypeStruct((2, NC, N_PAD, D2), jnp.float32),
    scratch_types=[
        pltpu.VMEM((AR, AW), jnp.int32),
        pltpu.VMEM((AR, AW), jnp.int32),
        pltpu.VMEM((AR, AW), jnp.float32),
        pltpu.VMEM((AW, D2), jnp.float32),
        pltpu.VMEM((TN, D2), jnp.float32),
        pltpu.VMEM_SHARED((N_PAD, D2), jnp.float32),
        pltpu.SemaphoreType.DMA,
    ],
    compiler_params=_SC_PARAMS,
)
def _aggr_sc(z0_hbm, z1_hbm, src_hbm, dst_hbm, p_hbm,
             num_hbm,
             srcv, dstv, pvf, rows, nbuf, num_sp, sem):
    c = lax.axis_index("c")
    s = lax.axis_index("s")
    tile = c * NS + s
    nbase = s * TN

    pltpu.sync_copy(src_hbm.at[tile], srcv)
    pltpu.sync_copy(dst_hbm.at[tile], dstv)
    pltpu.sync_copy(p_hbm.at[tile], pvf)

    zero = jnp.zeros((16,), jnp.float32)

    for h, zh_hbm in enumerate((z0_hbm, z1_hbm)):
        def zrow(i, _):
            for k in range(D2 // 16):
                rows[i, pl.ds(k * 16, 16)] = zero
            return ()

        lax.fori_loop(0, AW, zrow, ())

        def zslice(j, _):
            pltpu.sync_copy(rows, num_sp.at[pl.ds(nbase + j * AW, AW)])
            return ()

        lax.fori_loop(0, TN // AW, zslice, ())
        plsc.subcore_barrier()

        def erow(j, _):
            pltpu.async_copy(zh_hbm.at[srcv.at[j]], rows, sem).wait()
            for k in range(AW // 16):
                pvec = pvf[j, pl.ds(k * 16, 16)]
                for r in range(16):
                    pr = _bcast_lane(pvec, r)
                    for q in range(D2 // 16):
                        rr = k * 16 + r
                        rows[rr, pl.ds(q * 16, 16)] = (
                            rows[rr, pl.ds(q * 16, 16)] * pr)
            pltpu.sync_copy(rows, num_sp.at[dstv.at[j]], add=True)
            return ()

        lax.fori_loop(0, AR, erow, ())
        plsc.subcore_barrier()
        pltpu.sync_copy(num_sp.at[pl.ds(nbase, TN)], nbuf)
        pltpu.sync_copy(nbuf, num_hbm.at[h, c, pl.ds(nbase, TN)])


def kernel(node_id, edge_index, img_h, txt_h, emb_table, W_fc, a_attn):
    del img_h, txt_h
    h = jnp.take(emb_table, node_id, axis=0)
    wt = W_fc.T
    a2c = a_attn.reshape(2, D).T

    z, svals, smax = pl.pallas_call(
        _prep_tc,
        grid=(N // BLK,),
        in_specs=[
            pl.BlockSpec((BLK, D), lambda i: (i, 0)),
            pl.BlockSpec((D, D), lambda i: (0, 0)),
            pl.BlockSpec((D, 2), lambda i: (0, 0)),
        ],
        out_specs=[
            pl.BlockSpec((BLK, D), lambda i: (i, 0)),
            pl.BlockSpec((BLK, 2), lambda i: (i, 0)),
            pl.BlockSpec((1, 2), lambda i: (0, 0)),
        ],
        out_shape=[
            jax.ShapeDtypeStruct((N, D), jnp.float32),
            jax.ShapeDtypeStruct((N, 2), jnp.float32),
            jax.ShapeDtypeStruct((1, 2), jnp.float32),
        ],
    )(h, wt, a2c)

    s1 = svals[:, 0]
    s2 = svals[:, 1]
    shift = jnp.full((16,), smax[0, 0] + smax[0, 1], jnp.float32)

    src = edge_index[0].astype(jnp.int32)
    dst = edge_index[1].astype(jnp.int32)
    src_a = src.reshape(NT, AR, AW)
    dst_a = dst.reshape(NT, AR, AW)

    p, den = _edge_sc(s1, s2, src_a, dst_a, shift)
    z0 = z[:, :D2]
    z1 = z[:, D2:]
    num = _aggr_sc(z0, z1, src_a, dst_a, p)

    halves = []
    for h in range(2):
        halves.append(pl.pallas_call(
            _finish_tc,
            grid=(N // BLK,),
            in_specs=[
                pl.BlockSpec((BLK, D2), lambda i: (i, 0)),
                pl.BlockSpec((BLK, D2), lambda i: (i, 0)),
                pl.BlockSpec((BLK, 1), lambda i: (i, 0)),
                pl.BlockSpec((BLK, 1), lambda i: (i, 0)),
            ],
            out_specs=pl.BlockSpec((BLK, D2), lambda i: (i, 0)),
            out_shape=jax.ShapeDtypeStruct((N, D2), jnp.float32),
        )(num[h, 0, :N], num[h, 1, :N], den[0, :N, None], den[1, :N, None]))
    return jnp.concatenate(halves, axis=1)
